# Initial kernel scaffold; baseline (speedup 1.0000x reference)
import jax
import jax.numpy as jnp
from jax.experimental import pallas as pl


def kernel(boxes, W_w, W_h, W_cx, W_cy, W_xs, W_ys, W_x1, W_y1, W_x3, W_y3, W_cat, W_merge, W_colspan):
    raise NotImplementedError("write your pallas kernel here")


# speedup vs baseline: 4.2857x; 4.2857x over previous
import functools

import jax
import jax.numpy as jnp
from jax import lax
from jax.experimental import pallas as pl
from jax.experimental.pallas import tpu as pltpu
from jax.experimental.pallas import tpu_sc as plsc

VOCAB = 100000
BBOX_SIZE = 99998
D = 64
NF = 9
NT = 13
CHUNK = 128
L16 = 16


def _trunc_half(a):
    q = lax.shift_right_logical(jnp.abs(a), 1)
    return jnp.where(a < 0, -q, q)


def _sc_embed_build(n_tokens):
    nc, ns = 2, 16
    nw = nc * ns
    assert n_tokens % (nw * CHUNK) == 0
    tok_per_w = n_tokens // nw
    n_chunks = tok_per_w // CHUNK

    mesh = plsc.VectorSubcoreMesh(core_axis_name="c", subcore_axis_name="s")

    @functools.partial(
        pl.kernel,
        out_type=jax.ShapeDtypeStruct((n_tokens, 2 * D), jnp.float32),
        mesh=mesh,
        scratch_types=[
            pltpu.VMEM((NF, CHUNK), jnp.int32),
            pltpu.VMEM((NT, CHUNK), jnp.int32),
            pltpu.VMEM((NT, CHUNK, D), jnp.float32),
            pltpu.VMEM((CHUNK, 2 * D), jnp.float32),
            pltpu.SemaphoreType.DMA,
        ],
        compiler_params=pltpu.CompilerParams(use_tc_tiling_on_sc=False),
    )
    def sc_embed(boxes_t, w_w, w_h, w_cx, w_cy, w_xs, w_ys, w_x1, w_y1,
                 w_x3, w_y3, w_cat, w_merge, w_colspan, out,
                 fields, idxs, gbuf, ostage, sem):
        tables = (w_w, w_h, w_cx, w_cy, w_xs, w_ys, w_x1, w_y1, w_x3, w_y3,
                  w_cat, w_merge, w_colspan)
        wid = lax.axis_index("s") * nc + lax.axis_index("c")
        base0 = wid * tok_per_w

        def chunk_body(c):
            base = base0 + c * CHUNK
            pltpu.sync_copy(boxes_t.at[:, pl.ds(base, CHUNK)], fields)

            for g in range(CHUNK // L16):
                s = pl.ds(g * L16, L16)

                def fld(i):
                    v = fields[i, s]
                    return jnp.minimum(jnp.maximum(v, 0), VOCAB)

                cx, cy, w, h, xs, ys = (fld(i) for i in range(6))
                cat, mrg, csp = (fld(i) for i in range(6, 9))
                xa = _trunc_half(xs - BBOX_SIZE // 2)
                ya = _trunc_half(ys - BBOX_SIZE // 2)
                half_w = lax.shift_right_logical(w, 1)
                half_h = lax.shift_right_logical(h, 1)

                def bclip(v):
                    return jnp.minimum(jnp.maximum(v, 0), BBOX_SIZE)

                x1 = bclip(cx - half_w - xa)
                y1 = bclip(cy - half_h - ya)
                x3 = bclip(cx + half_w + xa)
                y3 = bclip(cy + half_h + ya)

                def vclip(v):
                    return jnp.minimum(v, VOCAB - 1)

                for t, v in enumerate((w, h, cx, cy, xs, ys)):
                    idxs[t, s] = vclip(v)
                for t, v in zip(range(6, 10), (x1, y1, x3, y3)):
                    idxs[t, s] = v
                for t, v in zip(range(10, 13), (cat, mrg, csp)):
                    idxs[t, s] = vclip(v)

            copies = [
                pltpu.async_copy(tables[t].at[idxs.at[t]], gbuf.at[t], sem)
                for t in range(NT)
            ]
            for cp in copies:
                cp.wait()

            def tok_body(tk):
                for q in range(D // L16):
                    s = pl.ds(q * L16, L16)
                    vb = gbuf[0, tk, s]
                    for t in range(1, 10):
                        vb = vb + gbuf[t, tk, s]
                    vp = gbuf[10, tk, s]
                    for t in range(11, 13):
                        vp = vp + gbuf[t, tk, s]
                    ostage[tk, s] = vb
                    ostage[tk, pl.ds(D + q * L16, L16)] = vp

            pl.loop(0, CHUNK)(tok_body)
            pltpu.sync_copy(ostage, out.at[pl.ds(base, CHUNK)])

        pl.loop(0, n_chunks)(chunk_body)

    return sc_embed


@jax.jit
def kernel(boxes, W_w, W_h, W_cx, W_cy, W_xs, W_ys, W_x1, W_y1, W_x3, W_y3,
           W_cat, W_merge, W_colspan):
    b, l, _ = boxes.shape
    n = b * l
    boxes_t = boxes.astype(jnp.int32).reshape(n, NF).T
    fn = _sc_embed_build(n)
    out = fn(boxes_t, W_w, W_h, W_cx, W_cy, W_xs, W_ys, W_x1, W_y1, W_x3,
             W_y3, W_cat, W_merge, W_colspan)
    return out.reshape(b, l, 2 * D)
